# Initial kernel scaffold; baseline (speedup 1.0000x reference)
import jax
import jax.numpy as jnp
from jax.experimental import pallas as pl


def kernel(x_m, x_d, gx_W, gx_as, gx_ad, gx_b, gy_W, gy_as, gy_ad, gy_b, cnn_x_w, cnn_x_b, cnn_y_w, cnn_y_b, mm_f_edges, dd_f_edges):
    raise NotImplementedError("write your pallas kernel here")


# speedup vs baseline: 1.0678x; 1.0678x over previous
import functools

import jax
import jax.numpy as jnp
from jax.experimental import pallas as pl
from jax.experimental.pallas import tpu as pltpu

N = 10000
F = 128
L = 3
E = 320000


def _mm_body(a_ref, w_ref, o_ref):
    o_ref[...] = jnp.dot(a_ref[...], w_ref[...],
                         preferred_element_type=jnp.float32)


def _pallas_matmul(a, w, bm=1000):
    m, k = a.shape
    k2, n = w.shape
    grid = (m // bm,)
    return pl.pallas_call(
        _mm_body,
        grid=grid,
        in_specs=[
            pl.BlockSpec((bm, k), lambda i: (i, 0)),
            pl.BlockSpec((k2, n), lambda i: (0, 0)),
        ],
        out_specs=pl.BlockSpec((bm, n), lambda i: (i, 0)),
        out_shape=jax.ShapeDtypeStruct((m, n), jnp.float32),
    )(a, w)


def _final_body(x_ref, y_ref, o_ref):
    o_ref[...] = jax.lax.dot_general(
        x_ref[...], y_ref[...], (((1,), (1,)), ((), ())),
        preferred_element_type=jnp.float32)


def _pallas_final(x, y, bm=400):
    return pl.pallas_call(
        _final_body,
        grid=(N // bm,),
        in_specs=[
            pl.BlockSpec((bm, F), lambda i: (i, 0)),
            pl.BlockSpec((N, F), lambda i: (0, 0)),
        ],
        out_specs=pl.BlockSpec((bm, N), lambda i: (i, 0)),
        out_shape=jax.ShapeDtypeStruct((N, N), jnp.float32),
    )(x, y)


def _gat_conv(x, src, dst, W, a_src, a_dst, b, n):
    h = _pallas_matmul(x, W.T)
    loop = jnp.arange(n, dtype=src.dtype)
    s = jnp.concatenate([src, loop])
    d = jnp.concatenate([dst, loop])
    als = (h * a_src[None, :]).sum(-1)
    ald = (h * a_dst[None, :]).sum(-1)
    alpha = jax.nn.leaky_relu(als[s] + ald[d], 0.2)
    amax = jax.ops.segment_max(alpha, d, num_segments=n)
    amax = jnp.where(jnp.isfinite(amax), amax, 0.0)
    ex = jnp.exp(alpha - amax[d])
    denom = jax.ops.segment_sum(ex, d, num_segments=n)
    coef = ex / (denom[d] + 1e-16)
    out = jax.ops.segment_sum(h[s] * coef[:, None], d, num_segments=n)
    return out + b[None, :]


def _branch(x0, edges, gW, gas, gad, gb, cw, cb, n, f):
    src, dst = edges[0], edges[1]
    h1 = jax.nn.relu(_gat_conv(x0, src, dst, gW[0], gas[0], gad[0], gb[0], n))
    h2 = jax.nn.relu(_gat_conv(h1, src, dst, gW[1], gas[1], gad[1], gb[1], n))
    h3 = jax.nn.relu(_gat_conv(h2, src, dst, gW[2], gas[2], gad[2], gb[2], n))
    w0 = cw[..., 0]
    z = (_pallas_matmul(h1, w0[:, 0, :].T)
         + _pallas_matmul(h2, w0[:, 1, :].T)
         + _pallas_matmul(h3, w0[:, 2, :].T))
    return z + cb[None, :]


def kernel(x_m, x_d, gx_W, gx_as, gx_ad, gx_b, gy_W, gy_as, gy_ad, gy_b,
           cnn_x_w, cnn_x_b, cnn_y_w, cnn_y_b, mm_f_edges, dd_f_edges):
    x = _branch(x_m, mm_f_edges, gx_W, gx_as, gx_ad, gx_b, cnn_x_w, cnn_x_b, N, F)
    y = _branch(x_d, dd_f_edges, gy_W, gy_as, gy_ad, gy_b, cnn_y_w, cnn_y_b, N, F)
    return _pallas_final(x, y)


# speedup vs baseline: 4.6676x; 4.3711x over previous
import functools

import jax
import jax.numpy as jnp
from jax import lax
from jax.experimental import pallas as pl
from jax.experimental.pallas import tpu as pltpu
from jax.experimental.pallas import tpu_sc as plsc

N = 10000
F = 128
L = 3
E = 320000
NB = 2

NSC = 16
TPB = 640
NP = NSC * TPB
CP = 1280
NCHUNK = E // CP
SB = 64
NSUB = CP // SB

_MESH = plsc.VectorSubcoreMesh(core_axis_name="c", subcore_axis_name="s")


def _leaky(x):
    return jnp.where(x >= 0, x, 0.2 * x)


def _bucket_body(src_hbm, dst_hbm, lsrc_hbm, ldst_hbm, cnt_hbm,
                 srcv, dstv, osrc, odst, cntv):
    c = lax.axis_index("c")
    s = lax.axis_index("s")
    lo = s * TPB
    hi = lo + TPB
    lane = lax.iota(jnp.int32, 16)
    zeros = jnp.zeros((16,), jnp.int32)

    def _zero(i, _):
        osrc[pl.ds(i * 16, 16)] = zeros
        odst[pl.ds(i * 16, 16)] = zeros
        return 0
    lax.fori_loop(0, (CP + 32) // 16, _zero, 0)

    def _chunk(g, _):
        pltpu.sync_copy(src_hbm.at[c, pl.ds(g * CP, CP)], srcv)
        pltpu.sync_copy(dst_hbm.at[c, pl.ds(g * CP, CP)], dstv)

        def _group(i, cl):
            s16 = srcv[pl.ds(i * 16, 16)]
            d16 = dstv[pl.ds(i * 16, 16)]
            m = (d16 >= lo) & (d16 < hi)
            rank = plsc.cumsum(m.astype(jnp.int32))
            pos = jnp.where(m, cl + rank - 1, CP + 16 + lane)
            plsc.store_scatter(osrc, [pos], s16)
            plsc.store_scatter(odst, [pos], d16)
            pc = plsc.all_reduce_population_count(m)
            return cl + pc[0]

        cl = lax.fori_loop(0, CP // 16, _group, jnp.int32(0))
        pltpu.sync_copy(osrc.at[pl.ds(0, CP)], lsrc_hbm.at[c, s, g])
        pltpu.sync_copy(odst.at[pl.ds(0, CP)], ldst_hbm.at[c, s, g])
        cntv[g] = jnp.full((16,), cl, jnp.int32)
        return 0

    lax.fori_loop(0, NCHUNK, _chunk, 0)
    pltpu.sync_copy(cntv, cnt_hbm.at[c, s])


_bucket = functools.partial(
    pl.kernel,
    out_type=(
        jax.ShapeDtypeStruct((NB, NSC, NCHUNK, CP), jnp.int32),
        jax.ShapeDtypeStruct((NB, NSC, NCHUNK, CP), jnp.int32),
        jax.ShapeDtypeStruct((NB, NSC, 256, 16), jnp.int32),
    ),
    mesh=_MESH,
    scratch_types=[
        pltpu.VMEM((CP,), jnp.int32),
        pltpu.VMEM((CP,), jnp.int32),
        pltpu.VMEM((CP + 32,), jnp.int32),
        pltpu.VMEM((CP + 32,), jnp.int32),
        pltpu.VMEM((256, 16), jnp.int32),
    ],
    compiler_params=pltpu.CompilerParams(needs_layout_passes=False),
)(_bucket_body)


def _wgt_body(als_hbm, ald_hbm, mh_hbm, lsrc_hbm, ldst_hbm, cnt_hbm,
              wl_hbm, selfw_hbm,
              alsv, aldv, mhv, cntv, srcv, dstv, wslot, swv):
    c = lax.axis_index("c")
    s = lax.axis_index("s")
    base = s * TPB

    pltpu.sync_copy(als_hbm.at[c], alsv)
    pltpu.sync_copy(ald_hbm.at[c], aldv)
    pltpu.sync_copy(mh_hbm.at[c], mhv)
    pltpu.sync_copy(cnt_hbm.at[c, s], cntv)
    mh = mhv[0][0]

    def _sw(g, _):
        rbase = base + g * 16

        @pl.when(rbase < N)
        def _real():
            a16 = alsv[pl.ds(rbase, 16)]
            d16 = aldv[pl.ds(rbase, 16)]
            swv[pl.ds(g * 16, 16)] = jnp.exp(_leaky(a16 + d16) - mh)

        return 0
    lax.fori_loop(0, TPB // 16, _sw, 0)
    pltpu.sync_copy(swv, selfw_hbm.at[c, pl.ds(base, TPB)])

    def _slot(g, _):
        pltpu.sync_copy(lsrc_hbm.at[c, s, g], srcv)
        pltpu.sync_copy(ldst_hbm.at[c, s, g], dstv)
        cnt = cntv[g][0]

        def _g16(i, _):
            s16 = srcv[pl.ds(i * 16, 16)]
            d16 = dstv[pl.ds(i * 16, 16)]
            aw = plsc.load_gather(alsv, [s16])
            dw = plsc.load_gather(aldv, [d16])
            wslot[pl.ds(i * 16, 16)] = jnp.exp(_leaky(aw + dw) - mh)
            return 0
        lax.fori_loop(0, (cnt + 15) // 16, _g16, 0)
        pltpu.sync_copy(wslot, wl_hbm.at[c, s, g])
        return 0

    lax.fori_loop(0, NCHUNK, _slot, 0)


_wgt = functools.partial(
    pl.kernel,
    out_type=(
        jax.ShapeDtypeStruct((NB, NSC, NCHUNK, CP), jnp.float32),
        jax.ShapeDtypeStruct((NB, NP), jnp.float32),
    ),
    mesh=_MESH,
    scratch_types=[
        pltpu.VMEM((N,), jnp.float32),
        pltpu.VMEM((N,), jnp.float32),
        pltpu.VMEM((8, 16), jnp.float32),
        pltpu.VMEM((256, 16), jnp.int32),
        pltpu.VMEM((CP,), jnp.int32),
        pltpu.VMEM((CP,), jnp.int32),
        pltpu.VMEM((CP,), jnp.float32),
        pltpu.VMEM((TPB,), jnp.float32),
    ],
    compiler_params=pltpu.CompilerParams(needs_layout_passes=False),
)(_wgt_body)


def _agg_body(hf_hbm, selfw_hbm, lsrc_hbm, ldst_hbm, wl_hbm, cnt_hbm,
              num_hbm, den_hbm,
              cntv, srcv, dstv, wslot, adjv, rows, denstage, acc, dens):
    c = lax.axis_index("c")
    s = lax.axis_index("s")
    base = s * TPB
    cn = c * N
    lane = lax.iota(jnp.int32, 16)
    fz = jnp.zeros((16,), jnp.float32)

    pltpu.sync_copy(cnt_hbm.at[c, s], cntv)
    pltpu.sync_copy(selfw_hbm.at[c, pl.ds(base, TPB)], denstage)

    def _zd(i, _):
        dens[i] = 0.0
        return 0
    lax.fori_loop(0, TPB, _zd, 0)

    def _init(g, _):
        rbase = base + g * 16

        @pl.when(rbase < N)
        def _real():
            pltpu.sync_copy(hf_hbm.at[pl.ds(cn + rbase, 16)],
                            rows.at[pl.ds(0, 16)])

            def _row(r, _):
                row = g * 16 + r
                wvec = plsc.load_gather(denstage,
                                        [jnp.full((16,), row, jnp.int32)])
                for j in range(8):
                    sl = pl.ds(j * 16, 16)
                    acc[row, sl] = wvec * rows[r, sl]
                return 0
            lax.fori_loop(0, 16, _row, 0)

        return 0
    lax.fori_loop(0, TPB // 16, _init, 0)

    def _slot(g, _):
        pltpu.sync_copy(lsrc_hbm.at[c, s, g], srcv)
        pltpu.sync_copy(ldst_hbm.at[c, s, g], dstv)
        pltpu.sync_copy(wl_hbm.at[c, s, g], wslot)
        cnt = cntv[g][0]
        nsub = (cnt + (SB - 1)) // SB

        def _sub(k, _):
            eb = k * SB
            for gg in range(SB // 16):
                s16 = srcv[pl.ds(eb + gg * 16, 16)]
                adjv[pl.ds(gg * 16, 16)] = s16 + cn
            pltpu.sync_copy(hf_hbm.at[adjv], rows)

            m = jnp.minimum(cnt - eb, SB)

            def _grp(g2, _):
                gb = g2 * 16
                d16 = dstv[pl.ds(eb + gb, 16)]
                w16 = wslot[pl.ds(eb + gb, 16)]
                w16 = jnp.where(gb + lane < m, w16, 0.0)
                dl16 = jnp.clip(d16 - base, 0, TPB - 1)
                for kk in range(16):
                    d_loc = dl16[kk]
                    w = w16[kk]
                    dens[d_loc] = dens[d_loc] + w
                    row_i = gb + kk
                    for j in range(8):
                        sl = pl.ds(j * 16, 16)
                        acc[d_loc, sl] = acc[d_loc, sl] + w * rows[row_i, sl]
                return 0
            lax.fori_loop(0, (m + 15) // 16, _grp, 0)
            return 0

        lax.fori_loop(0, nsub, _sub, 0)
        return 0

    lax.fori_loop(0, NCHUNK, _slot, 0)

    def _wout(g, _):
        vec = fz
        for k in range(16):
            vec = jnp.where(lane == k, dens[g * 16 + k], vec)
        denstage[pl.ds(g * 16, 16)] = denstage[pl.ds(g * 16, 16)] + vec
        return 0
    lax.fori_loop(0, TPB // 16, _wout, 0)

    pltpu.sync_copy(acc, num_hbm.at[c, pl.ds(base, TPB)])
    pltpu.sync_copy(denstage, den_hbm.at[c, pl.ds(base, TPB)])


_agg = functools.partial(
    pl.kernel,
    out_type=(
        jax.ShapeDtypeStruct((NB, NP, F), jnp.float32),
        jax.ShapeDtypeStruct((NB, NP), jnp.float32),
    ),
    mesh=_MESH,
    scratch_types=[
        pltpu.VMEM((256, 16), jnp.int32),
        pltpu.VMEM((CP,), jnp.int32),
        pltpu.VMEM((CP,), jnp.int32),
        pltpu.VMEM((CP,), jnp.float32),
        pltpu.VMEM((SB,), jnp.int32),
        pltpu.VMEM((SB, F), jnp.float32),
        pltpu.VMEM((TPB,), jnp.float32),
        pltpu.VMEM((TPB, F), jnp.float32),
        pltpu.SMEM((TPB,), jnp.float32),
    ],
    compiler_params=pltpu.CompilerParams(needs_layout_passes=False),
)(_agg_body)


BM = 1000
NBLK = N // BM


def _layer0_body(act_ref, w_ref, asrc_ref, adst_ref,
                 h_ref, als_ref, ald_ref, mh_ref, msc):
    i = pl.program_id(1)
    a = act_ref[0]
    h = lax.dot_general(a, w_ref[0], (((1,), (1,)), ((), ())),
                        preferred_element_type=jnp.float32)
    h_ref[0] = h
    als = h @ asrc_ref[0, 0]
    ald = h @ adst_ref[0, 0]
    als_ref[0, :, 0] = als
    ald_ref[0, :, 0] = ald
    ms = jnp.max(als)
    md = jnp.max(ald)

    @pl.when(i == 0)
    def _():
        msc[0] = ms
        msc[1] = md

    @pl.when(i > 0)
    def _():
        msc[0] = jnp.maximum(msc[0], ms)
        msc[1] = jnp.maximum(msc[1], md)

    @pl.when(i == NBLK - 1)
    def _():
        mh_ref[0] = jnp.full((8, 16), _leaky(msc[0] + msc[1]), jnp.float32)


def _layerN_body(num_ref, den_ref, bprev_ref, w_ref, asrc_ref, adst_ref,
                 act_ref, h_ref, als_ref, ald_ref, mh_ref, msc):
    i = pl.program_id(1)
    a = jax.nn.relu(num_ref[0] / (den_ref[0] + 1e-16) + bprev_ref[0])
    act_ref[0] = a
    h = lax.dot_general(a, w_ref[0], (((1,), (1,)), ((), ())),
                        preferred_element_type=jnp.float32)
    h_ref[0] = h
    als = h @ asrc_ref[0, 0]
    ald = h @ adst_ref[0, 0]
    als_ref[0, :, 0] = als
    ald_ref[0, :, 0] = ald
    ms = jnp.max(als)
    md = jnp.max(ald)

    @pl.when(i == 0)
    def _():
        msc[0] = ms
        msc[1] = md

    @pl.when(i > 0)
    def _():
        msc[0] = jnp.maximum(msc[0], ms)
        msc[1] = jnp.maximum(msc[1], md)

    @pl.when(i == NBLK - 1)
    def _():
        mh_ref[0] = jnp.full((8, 16), _leaky(msc[0] + msc[1]), jnp.float32)


_common_out = (
    jax.ShapeDtypeStruct((NB, N, F), jnp.float32),
    jax.ShapeDtypeStruct((NB, N, 1), jnp.float32),
    jax.ShapeDtypeStruct((NB, N, 1), jnp.float32),
    jax.ShapeDtypeStruct((NB, 8, 16), jnp.float32),
)
_common_out_specs = [
    pl.BlockSpec((1, BM, F), lambda b, i: (b, i, 0)),
    pl.BlockSpec((1, BM, 1), lambda b, i: (b, i, 0)),
    pl.BlockSpec((1, BM, 1), lambda b, i: (b, i, 0)),
    pl.BlockSpec((1, 8, 16), lambda b, i: (b, 0, 0)),
]
_w_specs = [
    pl.BlockSpec((1, F, F), lambda b, i: (b, 0, 0)),
    pl.BlockSpec((1, 1, F), lambda b, i: (b, 0, 0)),
    pl.BlockSpec((1, 1, F), lambda b, i: (b, 0, 0)),
]


def _layer0(act, W, asrc, adst):
    return pl.pallas_call(
        _layer0_body,
        grid=(NB, NBLK),
        in_specs=[pl.BlockSpec((1, BM, F), lambda b, i: (b, i, 0))] + _w_specs,
        out_specs=_common_out_specs,
        out_shape=_common_out,
        scratch_shapes=[pltpu.SMEM((2,), jnp.float32)],
    )(act, W, asrc, adst)


def _layerN(num, den, bprev, W, asrc, adst):
    return pl.pallas_call(
        _layerN_body,
        grid=(NB, NBLK),
        in_specs=[
            pl.BlockSpec((1, BM, F), lambda b, i: (b, i, 0)),
            pl.BlockSpec((1, BM, 1), lambda b, i: (b, i, 0)),
            pl.BlockSpec((1, 1, F), lambda b, i: (b, 0, 0)),
        ] + _w_specs,
        out_specs=[pl.BlockSpec((1, BM, F), lambda b, i: (b, i, 0))]
        + _common_out_specs,
        out_shape=(jax.ShapeDtypeStruct((NB, N, F), jnp.float32),)
        + _common_out,
        scratch_shapes=[pltpu.SMEM((2,), jnp.float32)],
    )(num, den, bprev, W, asrc, adst)


def _combine_body(a1_ref, a2_ref, num_ref, den_ref, b2_ref, cwt_ref, cb_ref,
                  z_ref):
    a3 = jax.nn.relu(num_ref[0] / (den_ref[0] + 1e-16) + b2_ref[0])
    z = (jnp.dot(a1_ref[0], cwt_ref[0, 0], preferred_element_type=jnp.float32)
         + jnp.dot(a2_ref[0], cwt_ref[0, 1], preferred_element_type=jnp.float32)
         + jnp.dot(a3, cwt_ref[0, 2], preferred_element_type=jnp.float32))
    z_ref[0] = z + cb_ref[0]


def _combine(a1, a2, num, den, b2, cwt, cb):
    return pl.pallas_call(
        _combine_body,
        grid=(NB, NBLK),
        in_specs=[
            pl.BlockSpec((1, BM, F), lambda b, i: (b, i, 0)),
            pl.BlockSpec((1, BM, F), lambda b, i: (b, i, 0)),
            pl.BlockSpec((1, BM, F), lambda b, i: (b, i, 0)),
            pl.BlockSpec((1, BM, 1), lambda b, i: (b, i, 0)),
            pl.BlockSpec((1, 1, F), lambda b, i: (b, 0, 0)),
            pl.BlockSpec((1, L, F, F), lambda b, i: (b, 0, 0, 0)),
            pl.BlockSpec((1, 1, F), lambda b, i: (b, 0, 0)),
        ],
        out_specs=pl.BlockSpec((1, BM, F), lambda b, i: (b, i, 0)),
        out_shape=jax.ShapeDtypeStruct((NB, N, F), jnp.float32),
    )(a1, a2, num, den, b2, cwt, cb)


def _final_body(x_ref, y_ref, o_ref):
    o_ref[...] = lax.dot_general(
        x_ref[...], y_ref[...], (((1,), (1,)), ((), ())),
        preferred_element_type=jnp.float32)


def _final(x, y, bm=400):
    return pl.pallas_call(
        _final_body,
        grid=(N // bm,),
        in_specs=[
            pl.BlockSpec((bm, F), lambda i: (i, 0)),
            pl.BlockSpec((N, F), lambda i: (0, 0)),
        ],
        out_specs=pl.BlockSpec((bm, N), lambda i: (i, 0)),
        out_shape=jax.ShapeDtypeStruct((N, N), jnp.float32),
    )(x, y)


def kernel(x_m, x_d, gx_W, gx_as, gx_ad, gx_b, gy_W, gy_as, gy_ad, gy_b,
           cnn_x_w, cnn_x_b, cnn_y_w, cnn_y_b, mm_f_edges, dd_f_edges):
    act0 = jnp.stack([x_m, x_d])
    W = jnp.stack([gx_W, gy_W])
    asrc = jnp.stack([gx_as, gy_as])[:, :, None, :]
    adst = jnp.stack([gx_ad, gy_ad])[:, :, None, :]
    bias = jnp.stack([gx_b, gy_b])[:, :, None, :]
    src = jnp.stack([mm_f_edges[0], dd_f_edges[0]])
    dst = jnp.stack([mm_f_edges[1], dd_f_edges[1]])
    cwt = jnp.stack([
        jnp.transpose(cnn_x_w[..., 0], (1, 2, 0)),
        jnp.transpose(cnn_y_w[..., 0], (1, 2, 0)),
    ])
    cb = jnp.stack([cnn_x_b, cnn_y_b])[:, None, :]

    lsrc, ldst, cnt = _bucket(src, dst)

    acts = []
    num = den = None
    for l in range(L):
        if l == 0:
            h, als, ald, mh = _layer0(act0, W[:, 0], asrc[:, 0], adst[:, 0])
        else:
            act, h, als, ald, mh = _layerN(
                num, den[..., None], bias[:, l - 1], W[:, l],
                asrc[:, l], adst[:, l])
            acts.append(act)
        hf = jnp.reshape(h, (NB * N, F))
        wl, selfw = _wgt(jnp.reshape(als, (NB, N)), jnp.reshape(ald, (NB, N)),
                         mh, lsrc, ldst, cnt)
        num, den = _agg(hf, selfw, lsrc, ldst, wl, cnt)
        num = num[:, :N]
        den = den[:, :N]

    z = _combine(acts[0], acts[1], num, den[..., None],
                 bias[:, L - 1], cwt, cb)
    return _final(z[0], z[1])


# speedup vs baseline: 7.5976x; 1.6277x over previous
import functools

import jax
import jax.numpy as jnp
from jax import lax
from jax.experimental import pallas as pl
from jax.experimental.pallas import tpu as pltpu
from jax.experimental.pallas import tpu_sc as plsc

N = 10000
F = 128
L = 3
E = 320000
NB = 2

NSC = 16
TPB = 640
NP = NSC * TPB
CP = 1280
NCHUNK = E // CP
SB = 32
NSUB = CP // SB

_MESH = plsc.VectorSubcoreMesh(core_axis_name="c", subcore_axis_name="s")


def _leaky(x):
    return jnp.where(x >= 0, x, 0.2 * x)


def _bucket_body(src_hbm, dst_hbm, lsrc_hbm, ldst_hbm, cnt_hbm,
                 srcv, dstv, osrc, odst, cntv):
    c = lax.axis_index("c")
    s = lax.axis_index("s")
    lo = s * TPB
    hi = lo + TPB
    lane = lax.iota(jnp.int32, 16)
    zeros = jnp.zeros((16,), jnp.int32)

    def _zero(i, _):
        osrc[pl.ds(i * 16, 16)] = zeros
        odst[pl.ds(i * 16, 16)] = zeros
        return 0
    lax.fori_loop(0, (CP + 32) // 16, _zero, 0)

    def _chunk(g, _):
        pltpu.sync_copy(src_hbm.at[c, pl.ds(g * CP, CP)], srcv)
        pltpu.sync_copy(dst_hbm.at[c, pl.ds(g * CP, CP)], dstv)

        def _group(i, cl):
            s16 = srcv[pl.ds(i * 16, 16)]
            d16 = dstv[pl.ds(i * 16, 16)]
            m = (d16 >= lo) & (d16 < hi)
            rank = plsc.cumsum(m.astype(jnp.int32))
            pos = jnp.where(m, cl + rank - 1, CP + 16 + lane)
            plsc.store_scatter(osrc, [pos], s16)
            plsc.store_scatter(odst, [pos], d16)
            pc = plsc.all_reduce_population_count(m)
            return cl + pc[0]

        cl = lax.fori_loop(0, CP // 16, _group, jnp.int32(0))
        pltpu.sync_copy(osrc.at[pl.ds(0, CP)], lsrc_hbm.at[c, s, g])
        pltpu.sync_copy(odst.at[pl.ds(0, CP)], ldst_hbm.at[c, s, g])
        cntv[g] = jnp.full((16,), cl, jnp.int32)
        return 0

    lax.fori_loop(0, NCHUNK, _chunk, 0)
    pltpu.sync_copy(cntv, cnt_hbm.at[c, s])


_bucket = functools.partial(
    pl.kernel,
    out_type=(
        jax.ShapeDtypeStruct((NB, NSC, NCHUNK, CP), jnp.int32),
        jax.ShapeDtypeStruct((NB, NSC, NCHUNK, CP), jnp.int32),
        jax.ShapeDtypeStruct((NB, NSC, 256, 16), jnp.int32),
    ),
    mesh=_MESH,
    scratch_types=[
        pltpu.VMEM((CP,), jnp.int32),
        pltpu.VMEM((CP,), jnp.int32),
        pltpu.VMEM((CP + 32,), jnp.int32),
        pltpu.VMEM((CP + 32,), jnp.int32),
        pltpu.VMEM((256, 16), jnp.int32),
    ],
    compiler_params=pltpu.CompilerParams(needs_layout_passes=False),
)(_bucket_body)


def _wgt_body(als_hbm, ald_hbm, mh_hbm, lsrc_hbm, ldst_hbm, cnt_hbm,
              wl_hbm, selfw_hbm,
              alsv, aldv, mhv, cntv, srcv, dstv, wslot, swv):
    c = lax.axis_index("c")
    s = lax.axis_index("s")
    base = s * TPB

    pltpu.sync_copy(als_hbm.at[c], alsv)
    pltpu.sync_copy(ald_hbm.at[c], aldv)
    pltpu.sync_copy(mh_hbm.at[c], mhv)
    pltpu.sync_copy(cnt_hbm.at[c, s], cntv)
    mh = mhv[0][0]

    def _sw(g, _):
        rbase = base + g * 16

        @pl.when(rbase < N)
        def _real():
            a16 = alsv[pl.ds(rbase, 16)]
            d16 = aldv[pl.ds(rbase, 16)]
            swv[pl.ds(g * 16, 16)] = jnp.exp(_leaky(a16 + d16) - mh)

        return 0
    lax.fori_loop(0, TPB // 16, _sw, 0)
    pltpu.sync_copy(swv, selfw_hbm.at[c, pl.ds(base, TPB)])

    def _slot(g, _):
        pltpu.sync_copy(lsrc_hbm.at[c, s, g], srcv)
        pltpu.sync_copy(ldst_hbm.at[c, s, g], dstv)
        cnt = cntv[g][0]

        def _g16(i, _):
            s16 = srcv[pl.ds(i * 16, 16)]
            d16 = dstv[pl.ds(i * 16, 16)]
            aw = plsc.load_gather(alsv, [s16])
            dw = plsc.load_gather(aldv, [d16])
            wslot[pl.ds(i * 16, 16)] = jnp.exp(_leaky(aw + dw) - mh)
            return 0
        lax.fori_loop(0, (cnt + 15) // 16, _g16, 0)
        pltpu.sync_copy(wslot, wl_hbm.at[c, s, g])
        return 0

    lax.fori_loop(0, NCHUNK, _slot, 0)


_wgt = functools.partial(
    pl.kernel,
    out_type=(
        jax.ShapeDtypeStruct((NB, NSC, NCHUNK, CP), jnp.float32),
        jax.ShapeDtypeStruct((NB, NP), jnp.float32),
    ),
    mesh=_MESH,
    scratch_types=[
        pltpu.VMEM((N,), jnp.float32),
        pltpu.VMEM((N,), jnp.float32),
        pltpu.VMEM((8, 16), jnp.float32),
        pltpu.VMEM((256, 16), jnp.int32),
        pltpu.VMEM((CP,), jnp.int32),
        pltpu.VMEM((CP,), jnp.int32),
        pltpu.VMEM((CP,), jnp.float32),
        pltpu.VMEM((TPB,), jnp.float32),
    ],
    compiler_params=pltpu.CompilerParams(needs_layout_passes=False),
)(_wgt_body)


def _agg_body(hf_hbm, selfw_hbm, lsrc_hbm, ldst_hbm, wl_hbm, cnt_hbm,
              num_hbm, den_hbm,
              cntv, srcv, dstv, wslot, adjv, rowsb, denstage, acc, dens,
              gsem0, gsem1):
    c = lax.axis_index("c")
    s = lax.axis_index("s")
    base = s * TPB
    cn = c * N
    lane = lax.iota(jnp.int32, 16)
    fz = jnp.zeros((16,), jnp.float32)

    pltpu.sync_copy(cnt_hbm.at[c, s], cntv)
    pltpu.sync_copy(selfw_hbm.at[c, pl.ds(base, TPB)], denstage)

    def _zd(i, _):
        dens[i] = 0.0
        return 0
    lax.fori_loop(0, TPB, _zd, 0)

    def _init(g, _):
        rbase = base + g * 16

        @pl.when(rbase < N)
        def _real():
            pltpu.sync_copy(hf_hbm.at[pl.ds(cn + rbase, 16)],
                            rowsb.at[0, pl.ds(0, 16)])

            def _row(r, _):
                row = g * 16 + r
                wvec = plsc.load_gather(denstage,
                                        [jnp.full((16,), row, jnp.int32)])
                for j in range(8):
                    sl = pl.ds(j * 16, 16)
                    acc[row, sl] = wvec * rowsb[0, r, sl]
                return 0
            lax.fori_loop(0, 16, _row, 0)

        return 0
    lax.fori_loop(0, TPB // 16, _init, 0)

    def _prep_adj(k, par):
        eb = k * SB
        for gg in range(SB // 16):
            s16 = srcv[pl.ds(eb + gg * 16, 16)]
            adjv[par, pl.ds(gg * 16, 16)] = s16 + cn

    def _fire_gather(par):
        gsem = gsem0 if par == 0 else gsem1
        pltpu.async_copy(hf_hbm.at[adjv.at[par]], rowsb.at[par], gsem)

    def _wait_gather(par):
        gsem = gsem0 if par == 0 else gsem1
        pltpu.make_async_copy(hf_hbm.at[adjv.at[par]], rowsb.at[par],
                              gsem).wait()

    def _proc_sub(k, par, cnt):
        eb = k * SB
        m = jnp.minimum(cnt - eb, SB)

        def _grp(g2, _):
            gb = g2 * 16
            d16 = dstv[pl.ds(eb + gb, 16)]
            w16 = wslot[pl.ds(eb + gb, 16)]
            w16 = jnp.where(gb + lane < m, w16, 0.0)
            dl16 = jnp.clip(d16 - base, 0, TPB - 1)
            for kk in range(16):
                d_loc = dl16[kk]
                w = w16[kk]
                dens[d_loc] = dens[d_loc] + w
                row_i = gb + kk
                for j in range(8):
                    sl = pl.ds(j * 16, 16)
                    acc[d_loc, sl] = acc[d_loc, sl] + w * rowsb[par, row_i, sl]
            return 0
        lax.fori_loop(0, (m + 15) // 16, _grp, 0)

    def _slot(g, _):
        pltpu.sync_copy(lsrc_hbm.at[c, s, g], srcv)
        pltpu.sync_copy(ldst_hbm.at[c, s, g], dstv)
        pltpu.sync_copy(wl_hbm.at[c, s, g], wslot)
        cnt = cntv[g][0]
        nsub = (cnt + (SB - 1)) // SB

        @pl.when(nsub > 0)
        def _has():
            _prep_adj(0, 0)
            _fire_gather(0)

            def _pair(k2, _):
                for par in range(2):
                    cur = k2 * 2 + par

                    @pl.when(cur < nsub)
                    def _do(cur=cur, par=par):
                        @pl.when(cur + 1 < nsub)
                        def _pf():
                            _prep_adj(cur + 1, 1 - par)
                            _fire_gather(1 - par)

                        _wait_gather(par)
                        _proc_sub(cur, par, cnt)
                return 0
            lax.fori_loop(0, (nsub + 1) // 2, _pair, 0)
        return 0

    lax.fori_loop(0, NCHUNK, _slot, 0)

    def _wout(g, _):
        vec = fz
        for k in range(16):
            vec = jnp.where(lane == k, dens[g * 16 + k], vec)
        denstage[pl.ds(g * 16, 16)] = denstage[pl.ds(g * 16, 16)] + vec
        return 0
    lax.fori_loop(0, TPB // 16, _wout, 0)

    pltpu.sync_copy(acc, num_hbm.at[c, pl.ds(base, TPB)])
    pltpu.sync_copy(denstage, den_hbm.at[c, pl.ds(base, TPB)])


_agg = functools.partial(
    pl.kernel,
    out_type=(
        jax.ShapeDtypeStruct((NB, NP, F), jnp.float32),
        jax.ShapeDtypeStruct((NB, NP), jnp.float32),
    ),
    mesh=_MESH,
    scratch_types=[
        pltpu.VMEM((256, 16), jnp.int32),
        pltpu.VMEM((CP,), jnp.int32),
        pltpu.VMEM((CP,), jnp.int32),
        pltpu.VMEM((CP,), jnp.float32),
        pltpu.VMEM((2, SB), jnp.int32),
        pltpu.VMEM((2, SB, F), jnp.float32),
        pltpu.VMEM((TPB,), jnp.float32),
        pltpu.VMEM((TPB, F), jnp.float32),
        pltpu.SMEM((TPB,), jnp.float32),
        pltpu.SemaphoreType.DMA,
        pltpu.SemaphoreType.DMA,
    ],
    compiler_params=pltpu.CompilerParams(needs_layout_passes=False),
)(_agg_body)


BM = 1000
NBLK = N // BM


def _layer0_body(act_ref, w_ref, asrc_ref, adst_ref,
                 h_ref, als_ref, ald_ref, mh_ref, msc):
    i = pl.program_id(1)
    a = act_ref[0]
    h = lax.dot_general(a, w_ref[0], (((1,), (1,)), ((), ())),
                        preferred_element_type=jnp.float32)
    h_ref[0] = h
    als = h @ asrc_ref[0, 0]
    ald = h @ adst_ref[0, 0]
    als_ref[0, :, 0] = als
    ald_ref[0, :, 0] = ald
    ms = jnp.max(als)
    md = jnp.max(ald)

    @pl.when(i == 0)
    def _():
        msc[0] = ms
        msc[1] = md

    @pl.when(i > 0)
    def _():
        msc[0] = jnp.maximum(msc[0], ms)
        msc[1] = jnp.maximum(msc[1], md)

    @pl.when(i == NBLK - 1)
    def _():
        mh_ref[0] = jnp.full((8, 16), _leaky(msc[0] + msc[1]), jnp.float32)


def _layerN_body(num_ref, den_ref, bprev_ref, w_ref, asrc_ref, adst_ref,
                 act_ref, h_ref, als_ref, ald_ref, mh_ref, msc):
    i = pl.program_id(1)
    a = jax.nn.relu(num_ref[0] / (den_ref[0] + 1e-16) + bprev_ref[0])
    act_ref[0] = a
    h = lax.dot_general(a, w_ref[0], (((1,), (1,)), ((), ())),
                        preferred_element_type=jnp.float32)
    h_ref[0] = h
    als = h @ asrc_ref[0, 0]
    ald = h @ adst_ref[0, 0]
    als_ref[0, :, 0] = als
    ald_ref[0, :, 0] = ald
    ms = jnp.max(als)
    md = jnp.max(ald)

    @pl.when(i == 0)
    def _():
        msc[0] = ms
        msc[1] = md

    @pl.when(i > 0)
    def _():
        msc[0] = jnp.maximum(msc[0], ms)
        msc[1] = jnp.maximum(msc[1], md)

    @pl.when(i == NBLK - 1)
    def _():
        mh_ref[0] = jnp.full((8, 16), _leaky(msc[0] + msc[1]), jnp.float32)


_common_out = (
    jax.ShapeDtypeStruct((NB, N, F), jnp.float32),
    jax.ShapeDtypeStruct((NB, N, 1), jnp.float32),
    jax.ShapeDtypeStruct((NB, N, 1), jnp.float32),
    jax.ShapeDtypeStruct((NB, 8, 16), jnp.float32),
)
_common_out_specs = [
    pl.BlockSpec((1, BM, F), lambda b, i: (b, i, 0)),
    pl.BlockSpec((1, BM, 1), lambda b, i: (b, i, 0)),
    pl.BlockSpec((1, BM, 1), lambda b, i: (b, i, 0)),
    pl.BlockSpec((1, 8, 16), lambda b, i: (b, 0, 0)),
]
_w_specs = [
    pl.BlockSpec((1, F, F), lambda b, i: (b, 0, 0)),
    pl.BlockSpec((1, 1, F), lambda b, i: (b, 0, 0)),
    pl.BlockSpec((1, 1, F), lambda b, i: (b, 0, 0)),
]


def _layer0(act, W, asrc, adst):
    return pl.pallas_call(
        _layer0_body,
        grid=(NB, NBLK),
        in_specs=[pl.BlockSpec((1, BM, F), lambda b, i: (b, i, 0))] + _w_specs,
        out_specs=_common_out_specs,
        out_shape=_common_out,
        scratch_shapes=[pltpu.SMEM((2,), jnp.float32)],
    )(act, W, asrc, adst)


def _layerN(num, den, bprev, W, asrc, adst):
    return pl.pallas_call(
        _layerN_body,
        grid=(NB, NBLK),
        in_specs=[
            pl.BlockSpec((1, BM, F), lambda b, i: (b, i, 0)),
            pl.BlockSpec((1, BM, 1), lambda b, i: (b, i, 0)),
            pl.BlockSpec((1, 1, F), lambda b, i: (b, 0, 0)),
        ] + _w_specs,
        out_specs=[pl.BlockSpec((1, BM, F), lambda b, i: (b, i, 0))]
        + _common_out_specs,
        out_shape=(jax.ShapeDtypeStruct((NB, N, F), jnp.float32),)
        + _common_out,
        scratch_shapes=[pltpu.SMEM((2,), jnp.float32)],
    )(num, den, bprev, W, asrc, adst)


def _combine_body(a1_ref, a2_ref, num_ref, den_ref, b2_ref, cwt_ref, cb_ref,
                  z_ref):
    a3 = jax.nn.relu(num_ref[0] / (den_ref[0] + 1e-16) + b2_ref[0])
    z = (jnp.dot(a1_ref[0], cwt_ref[0, 0], preferred_element_type=jnp.float32)
         + jnp.dot(a2_ref[0], cwt_ref[0, 1], preferred_element_type=jnp.float32)
         + jnp.dot(a3, cwt_ref[0, 2], preferred_element_type=jnp.float32))
    z_ref[0] = z + cb_ref[0]


def _combine(a1, a2, num, den, b2, cwt, cb):
    return pl.pallas_call(
        _combine_body,
        grid=(NB, NBLK),
        in_specs=[
            pl.BlockSpec((1, BM, F), lambda b, i: (b, i, 0)),
            pl.BlockSpec((1, BM, F), lambda b, i: (b, i, 0)),
            pl.BlockSpec((1, BM, F), lambda b, i: (b, i, 0)),
            pl.BlockSpec((1, BM, 1), lambda b, i: (b, i, 0)),
            pl.BlockSpec((1, 1, F), lambda b, i: (b, 0, 0)),
            pl.BlockSpec((1, L, F, F), lambda b, i: (b, 0, 0, 0)),
            pl.BlockSpec((1, 1, F), lambda b, i: (b, 0, 0)),
        ],
        out_specs=pl.BlockSpec((1, BM, F), lambda b, i: (b, i, 0)),
        out_shape=jax.ShapeDtypeStruct((NB, N, F), jnp.float32),
    )(a1, a2, num, den, b2, cwt, cb)


def _final_body(x_ref, y_ref, o_ref):
    o_ref[...] = lax.dot_general(
        x_ref[...], y_ref[...], (((1,), (1,)), ((), ())),
        preferred_element_type=jnp.float32)


def _final(x, y, bm=400):
    return pl.pallas_call(
        _final_body,
        grid=(N // bm,),
        in_specs=[
            pl.BlockSpec((bm, F), lambda i: (i, 0)),
            pl.BlockSpec((N, F), lambda i: (0, 0)),
        ],
        out_specs=pl.BlockSpec((bm, N), lambda i: (i, 0)),
        out_shape=jax.ShapeDtypeStruct((N, N), jnp.float32),
    )(x, y)


def kernel(x_m, x_d, gx_W, gx_as, gx_ad, gx_b, gy_W, gy_as, gy_ad, gy_b,
           cnn_x_w, cnn_x_b, cnn_y_w, cnn_y_b, mm_f_edges, dd_f_edges):
    act0 = jnp.stack([x_m, x_d])
    W = jnp.stack([gx_W, gy_W])
    asrc = jnp.stack([gx_as, gy_as])[:, :, None, :]
    adst = jnp.stack([gx_ad, gy_ad])[:, :, None, :]
    bias = jnp.stack([gx_b, gy_b])[:, :, None, :]
    src = jnp.stack([mm_f_edges[0], dd_f_edges[0]])
    dst = jnp.stack([mm_f_edges[1], dd_f_edges[1]])
    cwt = jnp.stack([
        jnp.transpose(cnn_x_w[..., 0], (1, 2, 0)),
        jnp.transpose(cnn_y_w[..., 0], (1, 2, 0)),
    ])
    cb = jnp.stack([cnn_x_b, cnn_y_b])[:, None, :]

    lsrc, ldst, cnt = _bucket(src, dst)

    acts = []
    num = den = None
    for l in range(L):
        if l == 0:
            h, als, ald, mh = _layer0(act0, W[:, 0], asrc[:, 0], adst[:, 0])
        else:
            act, h, als, ald, mh = _layerN(
                num, den[..., None], bias[:, l - 1], W[:, l],
                asrc[:, l], adst[:, l])
            acts.append(act)
        hf = jnp.reshape(h, (NB * N, F))
        wl, selfw = _wgt(jnp.reshape(als, (NB, N)), jnp.reshape(ald, (NB, N)),
                         mh, lsrc, ldst, cnt)
        num, den = _agg(hf, selfw, lsrc, ldst, wl, cnt)
        num = num[:, :N]
        den = den[:, :N]

    z = _combine(acts[0], acts[1], num, den[..., None],
                 bias[:, L - 1], cwt, cb)
    return _final(z[0], z[1])


# speedup vs baseline: 8.0146x; 1.0549x over previous
import functools

import jax
import jax.numpy as jnp
from jax import lax
from jax.experimental import pallas as pl
from jax.experimental.pallas import tpu as pltpu
from jax.experimental.pallas import tpu_sc as plsc

N = 10000
F = 128
L = 3
E = 320000
NB = 2

NSC = 16
TPB = 640
NP = NSC * TPB
CP = 1280
NCHUNK = E // CP
SB = 32
NSUB = CP // SB
TIER = 256

_MESH = plsc.VectorSubcoreMesh(core_axis_name="c", subcore_axis_name="s")


def _leaky(x):
    return jnp.where(x >= 0, x, 0.2 * x)


def _bucket_body(src_hbm, dst_hbm, lsrc_hbm, ldst_hbm, cnt_hbm,
                 srcv, dstv, osrc, odst, cntv):
    c = lax.axis_index("c")
    s = lax.axis_index("s")
    lo = s * TPB
    hi = lo + TPB
    lane = lax.iota(jnp.int32, 16)
    zeros = jnp.zeros((16,), jnp.int32)

    def _zero(i, _):
        osrc[pl.ds(i * 16, 16)] = zeros
        odst[pl.ds(i * 16, 16)] = zeros
        return 0
    lax.fori_loop(0, (CP + 32) // 16, _zero, 0)

    def _chunk(g, _):
        pltpu.sync_copy(src_hbm.at[c, pl.ds(g * CP, CP)], srcv)
        pltpu.sync_copy(dst_hbm.at[c, pl.ds(g * CP, CP)], dstv)

        def _group(i, cl):
            s16 = srcv[pl.ds(i * 16, 16)]
            d16 = dstv[pl.ds(i * 16, 16)]
            m = (d16 >= lo) & (d16 < hi)
            rank = plsc.cumsum(m.astype(jnp.int32))
            pos = jnp.where(m, cl + rank - 1, CP + 16 + lane)
            plsc.store_scatter(osrc, [pos], s16)
            plsc.store_scatter(odst, [pos], d16)
            pc = plsc.all_reduce_population_count(m)
            return cl + pc[0]

        cl = lax.fori_loop(0, CP // 16, _group, jnp.int32(0))
        pltpu.sync_copy(osrc.at[pl.ds(0, CP)], lsrc_hbm.at[c, s, g])
        pltpu.sync_copy(odst.at[pl.ds(0, CP)], ldst_hbm.at[c, s, g])
        cntv[g] = jnp.full((16,), cl, jnp.int32)
        return 0

    lax.fori_loop(0, NCHUNK, _chunk, 0)
    pltpu.sync_copy(cntv, cnt_hbm.at[c, s])


_bucket = functools.partial(
    pl.kernel,
    out_type=(
        jax.ShapeDtypeStruct((NB, NSC, NCHUNK, CP), jnp.int32),
        jax.ShapeDtypeStruct((NB, NSC, NCHUNK, CP), jnp.int32),
        jax.ShapeDtypeStruct((NB, NSC, 256, 16), jnp.int32),
    ),
    mesh=_MESH,
    scratch_types=[
        pltpu.VMEM((CP,), jnp.int32),
        pltpu.VMEM((CP,), jnp.int32),
        pltpu.VMEM((CP + 32,), jnp.int32),
        pltpu.VMEM((CP + 32,), jnp.int32),
        pltpu.VMEM((256, 16), jnp.int32),
    ],
    compiler_params=pltpu.CompilerParams(needs_layout_passes=False),
)(_bucket_body)


def _wgt_body(als_hbm, ald_hbm, mh_hbm, lsrc_hbm, ldst_hbm, cnt_hbm,
              wl_hbm, selfw_hbm,
              alsv, aldv, mhv, cntv, srcv, dstv, wslot, swv):
    c = lax.axis_index("c")
    s = lax.axis_index("s")
    base = s * TPB

    pltpu.sync_copy(als_hbm.at[c], alsv)
    pltpu.sync_copy(ald_hbm.at[c], aldv)
    pltpu.sync_copy(mh_hbm.at[c], mhv)
    pltpu.sync_copy(cnt_hbm.at[c, s], cntv)
    mh = mhv[0][0]

    def _sw(g, _):
        rbase = base + g * 16

        @pl.when(rbase < N)
        def _real():
            a16 = alsv[pl.ds(rbase, 16)]
            d16 = aldv[pl.ds(rbase, 16)]
            swv[pl.ds(g * 16, 16)] = jnp.exp(_leaky(a16 + d16) - mh)

        return 0
    lax.fori_loop(0, TPB // 16, _sw, 0)
    pltpu.sync_copy(swv, selfw_hbm.at[c, pl.ds(base, TPB)])

    def _slot(g, _):
        cnt = cntv[g][0]

        @pl.when(cnt <= TIER)
        def _ld_small():
            pltpu.sync_copy(lsrc_hbm.at[c, s, g, pl.ds(0, TIER)],
                            srcv.at[pl.ds(0, TIER)])
            pltpu.sync_copy(ldst_hbm.at[c, s, g, pl.ds(0, TIER)],
                            dstv.at[pl.ds(0, TIER)])

        @pl.when(cnt > TIER)
        def _ld_big():
            pltpu.sync_copy(lsrc_hbm.at[c, s, g], srcv)
            pltpu.sync_copy(ldst_hbm.at[c, s, g], dstv)

        def _g16(i, _):
            s16 = srcv[pl.ds(i * 16, 16)]
            d16 = dstv[pl.ds(i * 16, 16)]
            aw = plsc.load_gather(alsv, [s16])
            dw = plsc.load_gather(aldv, [d16])
            wslot[pl.ds(i * 16, 16)] = jnp.exp(_leaky(aw + dw) - mh)
            return 0
        lax.fori_loop(0, (cnt + 15) // 16, _g16, 0)

        @pl.when(cnt <= TIER)
        def _st_small():
            pltpu.sync_copy(wslot.at[pl.ds(0, TIER)],
                            wl_hbm.at[c, s, g, pl.ds(0, TIER)])

        @pl.when(cnt > TIER)
        def _st_big():
            pltpu.sync_copy(wslot, wl_hbm.at[c, s, g])
        return 0

    lax.fori_loop(0, NCHUNK, _slot, 0)


_wgt = functools.partial(
    pl.kernel,
    out_type=(
        jax.ShapeDtypeStruct((NB, NSC, NCHUNK, CP), jnp.float32),
        jax.ShapeDtypeStruct((NB, NP), jnp.float32),
    ),
    mesh=_MESH,
    scratch_types=[
        pltpu.VMEM((N,), jnp.float32),
        pltpu.VMEM((N,), jnp.float32),
        pltpu.VMEM((8, 16), jnp.float32),
        pltpu.VMEM((256, 16), jnp.int32),
        pltpu.VMEM((CP,), jnp.int32),
        pltpu.VMEM((CP,), jnp.int32),
        pltpu.VMEM((CP,), jnp.float32),
        pltpu.VMEM((TPB,), jnp.float32),
    ],
    compiler_params=pltpu.CompilerParams(needs_layout_passes=False),
)(_wgt_body)


def _agg_body(hf_hbm, selfw_hbm, lsrc_hbm, ldst_hbm, wl_hbm, cnt_hbm,
              num_hbm, den_hbm,
              cntv, srcv, dstv, wslot, adjv, rowsb, denstage, acc, dens,
              gsem0, gsem1):
    c = lax.axis_index("c")
    s = lax.axis_index("s")
    base = s * TPB
    cn = c * N
    lane = lax.iota(jnp.int32, 16)
    fz = jnp.zeros((16,), jnp.float32)

    pltpu.sync_copy(cnt_hbm.at[c, s], cntv)
    pltpu.sync_copy(selfw_hbm.at[c, pl.ds(base, TPB)], denstage)

    def _zd(i, _):
        dens[i] = 0.0
        return 0
    lax.fori_loop(0, TPB, _zd, 0)

    def _init(g, _):
        rbase = base + g * 16

        @pl.when(rbase < N)
        def _real():
            pltpu.sync_copy(hf_hbm.at[pl.ds(cn + rbase, 16)],
                            rowsb.at[0, pl.ds(0, 16)])

            def _row(r, _):
                row = g * 16 + r
                wvec = plsc.load_gather(denstage,
                                        [jnp.full((16,), row, jnp.int32)])
                for j in range(8):
                    sl = pl.ds(j * 16, 16)
                    acc[row, sl] = wvec * rowsb[0, r, sl]
                return 0
            lax.fori_loop(0, 16, _row, 0)

        return 0
    lax.fori_loop(0, TPB // 16, _init, 0)

    def _prep_adj(k, par):
        eb = k * SB
        for gg in range(SB // 16):
            s16 = srcv[pl.ds(eb + gg * 16, 16)]
            adjv[par, pl.ds(gg * 16, 16)] = s16 + cn

    def _fire_gather(par):
        gsem = gsem0 if par == 0 else gsem1
        pltpu.async_copy(hf_hbm.at[adjv.at[par]], rowsb.at[par], gsem)

    def _wait_gather(par):
        gsem = gsem0 if par == 0 else gsem1
        pltpu.make_async_copy(hf_hbm.at[adjv.at[par]], rowsb.at[par],
                              gsem).wait()

    def _proc_sub(k, par, cnt):
        eb = k * SB
        m = jnp.minimum(cnt - eb, SB)

        def _grp(g2, _):
            gb = g2 * 16
            d16 = dstv[pl.ds(eb + gb, 16)]
            w16 = wslot[pl.ds(eb + gb, 16)]
            w16 = jnp.where(gb + lane < m, w16, 0.0)
            dl16 = jnp.clip(d16 - base, 0, TPB - 1)
            for kk in range(16):
                d_loc = dl16[kk]
                w = w16[kk]
                dens[d_loc] = dens[d_loc] + w
                row_i = gb + kk
                for j in range(8):
                    sl = pl.ds(j * 16, 16)
                    acc[d_loc, sl] = acc[d_loc, sl] + w * rowsb[par, row_i, sl]
            return 0
        lax.fori_loop(0, (m + 15) // 16, _grp, 0)

    def _slot(g, _):
        cnt = cntv[g][0]

        @pl.when(cnt <= TIER)
        def _ld_small():
            pltpu.sync_copy(lsrc_hbm.at[c, s, g, pl.ds(0, TIER)],
                            srcv.at[pl.ds(0, TIER)])
            pltpu.sync_copy(ldst_hbm.at[c, s, g, pl.ds(0, TIER)],
                            dstv.at[pl.ds(0, TIER)])
            pltpu.sync_copy(wl_hbm.at[c, s, g, pl.ds(0, TIER)],
                            wslot.at[pl.ds(0, TIER)])

        @pl.when(cnt > TIER)
        def _ld_big():
            pltpu.sync_copy(lsrc_hbm.at[c, s, g], srcv)
            pltpu.sync_copy(ldst_hbm.at[c, s, g], dstv)
            pltpu.sync_copy(wl_hbm.at[c, s, g], wslot)

        nsub = (cnt + (SB - 1)) // SB

        @pl.when(nsub > 0)
        def _has():
            _prep_adj(0, 0)
            _fire_gather(0)

            def _pair(k2, _):
                for par in range(2):
                    cur = k2 * 2 + par

                    @pl.when(cur < nsub)
                    def _do(cur=cur, par=par):
                        @pl.when(cur + 1 < nsub)
                        def _pf():
                            _prep_adj(cur + 1, 1 - par)
                            _fire_gather(1 - par)

                        _wait_gather(par)
                        _proc_sub(cur, par, cnt)
                return 0
            lax.fori_loop(0, (nsub + 1) // 2, _pair, 0)
        return 0

    lax.fori_loop(0, NCHUNK, _slot, 0)

    def _wout(g, _):
        vec = fz
        for k in range(16):
            vec = jnp.where(lane == k, dens[g * 16 + k], vec)
        denstage[pl.ds(g * 16, 16)] = denstage[pl.ds(g * 16, 16)] + vec
        return 0
    lax.fori_loop(0, TPB // 16, _wout, 0)

    pltpu.sync_copy(acc, num_hbm.at[c, pl.ds(base, TPB)])
    pltpu.sync_copy(denstage, den_hbm.at[c, pl.ds(base, TPB)])


_agg = functools.partial(
    pl.kernel,
    out_type=(
        jax.ShapeDtypeStruct((NB, NP, F), jnp.float32),
        jax.ShapeDtypeStruct((NB, NP), jnp.float32),
    ),
    mesh=_MESH,
    scratch_types=[
        pltpu.VMEM((256, 16), jnp.int32),
        pltpu.VMEM((CP,), jnp.int32),
        pltpu.VMEM((CP,), jnp.int32),
        pltpu.VMEM((CP,), jnp.float32),
        pltpu.VMEM((2, SB), jnp.int32),
        pltpu.VMEM((2, SB, F), jnp.float32),
        pltpu.VMEM((TPB,), jnp.float32),
        pltpu.VMEM((TPB, F), jnp.float32),
        pltpu.SMEM((TPB,), jnp.float32),
        pltpu.SemaphoreType.DMA,
        pltpu.SemaphoreType.DMA,
    ],
    compiler_params=pltpu.CompilerParams(needs_layout_passes=False),
)(_agg_body)


BM = 1000
NBLK = N // BM


def _layer0_body(act_ref, w_ref, asrc_ref, adst_ref,
                 h_ref, als_ref, ald_ref, mh_ref, msc):
    i = pl.program_id(1)
    a = act_ref[0]
    h = lax.dot_general(a, w_ref[0], (((1,), (1,)), ((), ())),
                        preferred_element_type=jnp.float32)
    h_ref[0] = h
    als = h @ asrc_ref[0, 0]
    ald = h @ adst_ref[0, 0]
    als_ref[0, :, 0] = als
    ald_ref[0, :, 0] = ald
    ms = jnp.max(als)
    md = jnp.max(ald)

    @pl.when(i == 0)
    def _():
        msc[0] = ms
        msc[1] = md

    @pl.when(i > 0)
    def _():
        msc[0] = jnp.maximum(msc[0], ms)
        msc[1] = jnp.maximum(msc[1], md)

    @pl.when(i == NBLK - 1)
    def _():
        mh_ref[0] = jnp.full((8, 16), _leaky(msc[0] + msc[1]), jnp.float32)


def _layerN_body(num_ref, den_ref, bprev_ref, w_ref, asrc_ref, adst_ref,
                 act_ref, h_ref, als_ref, ald_ref, mh_ref, msc):
    i = pl.program_id(1)
    a = jax.nn.relu(num_ref[0] / (den_ref[0] + 1e-16) + bprev_ref[0])
    act_ref[0] = a
    h = lax.dot_general(a, w_ref[0], (((1,), (1,)), ((), ())),
                        preferred_element_type=jnp.float32)
    h_ref[0] = h
    als = h @ asrc_ref[0, 0]
    ald = h @ adst_ref[0, 0]
    als_ref[0, :, 0] = als
    ald_ref[0, :, 0] = ald
    ms = jnp.max(als)
    md = jnp.max(ald)

    @pl.when(i == 0)
    def _():
        msc[0] = ms
        msc[1] = md

    @pl.when(i > 0)
    def _():
        msc[0] = jnp.maximum(msc[0], ms)
        msc[1] = jnp.maximum(msc[1], md)

    @pl.when(i == NBLK - 1)
    def _():
        mh_ref[0] = jnp.full((8, 16), _leaky(msc[0] + msc[1]), jnp.float32)


_common_out = (
    jax.ShapeDtypeStruct((NB, N, F), jnp.float32),
    jax.ShapeDtypeStruct((NB, N, 1), jnp.float32),
    jax.ShapeDtypeStruct((NB, N, 1), jnp.float32),
    jax.ShapeDtypeStruct((NB, 8, 16), jnp.float32),
)
_common_out_specs = [
    pl.BlockSpec((1, BM, F), lambda b, i: (b, i, 0)),
    pl.BlockSpec((1, BM, 1), lambda b, i: (b, i, 0)),
    pl.BlockSpec((1, BM, 1), lambda b, i: (b, i, 0)),
    pl.BlockSpec((1, 8, 16), lambda b, i: (b, 0, 0)),
]
_w_specs = [
    pl.BlockSpec((1, F, F), lambda b, i: (b, 0, 0)),
    pl.BlockSpec((1, 1, F), lambda b, i: (b, 0, 0)),
    pl.BlockSpec((1, 1, F), lambda b, i: (b, 0, 0)),
]


def _layer0(act, W, asrc, adst):
    return pl.pallas_call(
        _layer0_body,
        grid=(NB, NBLK),
        in_specs=[pl.BlockSpec((1, BM, F), lambda b, i: (b, i, 0))] + _w_specs,
        out_specs=_common_out_specs,
        out_shape=_common_out,
        scratch_shapes=[pltpu.SMEM((2,), jnp.float32)],
    )(act, W, asrc, adst)


def _layerN(num, den, bprev, W, asrc, adst):
    return pl.pallas_call(
        _layerN_body,
        grid=(NB, NBLK),
        in_specs=[
            pl.BlockSpec((1, BM, F), lambda b, i: (b, i, 0)),
            pl.BlockSpec((1, BM, 1), lambda b, i: (b, i, 0)),
            pl.BlockSpec((1, 1, F), lambda b, i: (b, 0, 0)),
        ] + _w_specs,
        out_specs=[pl.BlockSpec((1, BM, F), lambda b, i: (b, i, 0))]
        + _common_out_specs,
        out_shape=(jax.ShapeDtypeStruct((NB, N, F), jnp.float32),)
        + _common_out,
        scratch_shapes=[pltpu.SMEM((2,), jnp.float32)],
    )(num, den, bprev, W, asrc, adst)


def _combine_body(a1_ref, a2_ref, num_ref, den_ref, b2_ref, cwt_ref, cb_ref,
                  z_ref):
    a3 = jax.nn.relu(num_ref[0] / (den_ref[0] + 1e-16) + b2_ref[0])
    z = (jnp.dot(a1_ref[0], cwt_ref[0, 0], preferred_element_type=jnp.float32)
         + jnp.dot(a2_ref[0], cwt_ref[0, 1], preferred_element_type=jnp.float32)
         + jnp.dot(a3, cwt_ref[0, 2], preferred_element_type=jnp.float32))
    z_ref[0] = z + cb_ref[0]


def _combine(a1, a2, num, den, b2, cwt, cb):
    return pl.pallas_call(
        _combine_body,
        grid=(NB, NBLK),
        in_specs=[
            pl.BlockSpec((1, BM, F), lambda b, i: (b, i, 0)),
            pl.BlockSpec((1, BM, F), lambda b, i: (b, i, 0)),
            pl.BlockSpec((1, BM, F), lambda b, i: (b, i, 0)),
            pl.BlockSpec((1, BM, 1), lambda b, i: (b, i, 0)),
            pl.BlockSpec((1, 1, F), lambda b, i: (b, 0, 0)),
            pl.BlockSpec((1, L, F, F), lambda b, i: (b, 0, 0, 0)),
            pl.BlockSpec((1, 1, F), lambda b, i: (b, 0, 0)),
        ],
        out_specs=pl.BlockSpec((1, BM, F), lambda b, i: (b, i, 0)),
        out_shape=jax.ShapeDtypeStruct((NB, N, F), jnp.float32),
    )(a1, a2, num, den, b2, cwt, cb)


def _final_body(x_ref, y_ref, o_ref):
    o_ref[...] = lax.dot_general(
        x_ref[...], y_ref[...], (((1,), (1,)), ((), ())),
        preferred_element_type=jnp.float32)


def _final(x, y, bm=400):
    return pl.pallas_call(
        _final_body,
        grid=(N // bm,),
        in_specs=[
            pl.BlockSpec((bm, F), lambda i: (i, 0)),
            pl.BlockSpec((N, F), lambda i: (0, 0)),
        ],
        out_specs=pl.BlockSpec((bm, N), lambda i: (i, 0)),
        out_shape=jax.ShapeDtypeStruct((N, N), jnp.float32),
    )(x, y)


def kernel(x_m, x_d, gx_W, gx_as, gx_ad, gx_b, gy_W, gy_as, gy_ad, gy_b,
           cnn_x_w, cnn_x_b, cnn_y_w, cnn_y_b, mm_f_edges, dd_f_edges):
    act0 = jnp.stack([x_m, x_d])
    W = jnp.stack([gx_W, gy_W])
    asrc = jnp.stack([gx_as, gy_as])[:, :, None, :]
    adst = jnp.stack([gx_ad, gy_ad])[:, :, None, :]
    bias = jnp.stack([gx_b, gy_b])[:, :, None, :]
    src = jnp.stack([mm_f_edges[0], dd_f_edges[0]])
    dst = jnp.stack([mm_f_edges[1], dd_f_edges[1]])
    cwt = jnp.stack([
        jnp.transpose(cnn_x_w[..., 0], (1, 2, 0)),
        jnp.transpose(cnn_y_w[..., 0], (1, 2, 0)),
    ])
    cb = jnp.stack([cnn_x_b, cnn_y_b])[:, None, :]

    lsrc, ldst, cnt = _bucket(src, dst)

    acts = []
    num = den = None
    for l in range(L):
        if l == 0:
            h, als, ald, mh = _layer0(act0, W[:, 0], asrc[:, 0], adst[:, 0])
        else:
            act, h, als, ald, mh = _layerN(
                num, den[..., None], bias[:, l - 1], W[:, l],
                asrc[:, l], adst[:, l])
            acts.append(act)
        hf = jnp.reshape(h, (NB * N, F))
        wl, selfw = _wgt(jnp.reshape(als, (NB, N)), jnp.reshape(ald, (NB, N)),
                         mh, lsrc, ldst, cnt)
        num, den = _agg(hf, selfw, lsrc, ldst, wl, cnt)
        num = num[:, :N]
        den = den[:, :N]

    z = _combine(acts[0], acts[1], num, den[..., None],
                 bias[:, L - 1], cwt, cb)
    return _final(z[0], z[1])


# speedup vs baseline: 8.1661x; 1.0189x over previous
import functools

import jax
import jax.numpy as jnp
from jax import lax
from jax.experimental import pallas as pl
from jax.experimental.pallas import tpu as pltpu
from jax.experimental.pallas import tpu_sc as plsc

N = 10000
F = 128
L = 3
E = 320000
NB = 2

NSC = 16
TPB = 640
NP = NSC * TPB
CP = 1280
NCHUNK = E // CP
SB = 32
NSUB = CP // SB
TIER = 128

_MESH = plsc.VectorSubcoreMesh(core_axis_name="c", subcore_axis_name="s")


def _leaky(x):
    return jnp.where(x >= 0, x, 0.2 * x)


def _bucket_body(src_hbm, dst_hbm, lsrc_hbm, ldst_hbm, cnt_hbm,
                 srcv, dstv, osrc, odst, cntv):
    c = lax.axis_index("c")
    s = lax.axis_index("s")
    lo = s * TPB
    hi = lo + TPB
    lane = lax.iota(jnp.int32, 16)
    zeros = jnp.zeros((16,), jnp.int32)

    def _zero(i, _):
        osrc[pl.ds(i * 16, 16)] = zeros
        odst[pl.ds(i * 16, 16)] = zeros
        return 0
    lax.fori_loop(0, (CP + 32) // 16, _zero, 0)

    def _chunk(g, _):
        pltpu.sync_copy(src_hbm.at[c, pl.ds(g * CP, CP)], srcv)
        pltpu.sync_copy(dst_hbm.at[c, pl.ds(g * CP, CP)], dstv)

        def _group(i, cl):
            s16 = srcv[pl.ds(i * 16, 16)]
            d16 = dstv[pl.ds(i * 16, 16)]
            m = (d16 >= lo) & (d16 < hi)
            rank = plsc.cumsum(m.astype(jnp.int32))
            pos = jnp.where(m, cl + rank - 1, CP + 16 + lane)
            plsc.store_scatter(osrc, [pos], s16)
            plsc.store_scatter(odst, [pos], d16)
            pc = plsc.all_reduce_population_count(m)
            return cl + pc[0]

        cl = lax.fori_loop(0, CP // 16, _group, jnp.int32(0))

        @pl.when(cl <= TIER)
        def _st_small():
            pltpu.sync_copy(osrc.at[pl.ds(0, TIER)],
                            lsrc_hbm.at[c, s, g, pl.ds(0, TIER)])
            pltpu.sync_copy(odst.at[pl.ds(0, TIER)],
                            ldst_hbm.at[c, s, g, pl.ds(0, TIER)])

        @pl.when(cl > TIER)
        def _st_big():
            pltpu.sync_copy(osrc.at[pl.ds(0, CP)], lsrc_hbm.at[c, s, g])
            pltpu.sync_copy(odst.at[pl.ds(0, CP)], ldst_hbm.at[c, s, g])

        cntv[g] = jnp.full((16,), cl, jnp.int32)
        return 0

    lax.fori_loop(0, NCHUNK, _chunk, 0)
    pltpu.sync_copy(cntv, cnt_hbm.at[c, s])


_bucket = functools.partial(
    pl.kernel,
    out_type=(
        jax.ShapeDtypeStruct((NB, NSC, NCHUNK, CP), jnp.int32),
        jax.ShapeDtypeStruct((NB, NSC, NCHUNK, CP), jnp.int32),
        jax.ShapeDtypeStruct((NB, NSC, 256, 16), jnp.int32),
    ),
    mesh=_MESH,
    scratch_types=[
        pltpu.VMEM((CP,), jnp.int32),
        pltpu.VMEM((CP,), jnp.int32),
        pltpu.VMEM((CP + 32,), jnp.int32),
        pltpu.VMEM((CP + 32,), jnp.int32),
        pltpu.VMEM((256, 16), jnp.int32),
    ],
    compiler_params=pltpu.CompilerParams(needs_layout_passes=False),
)(_bucket_body)


def _wgt_body(als_hbm, ald_hbm, mh_hbm, lsrc_hbm, ldst_hbm, cnt_hbm,
              wl_hbm, selfw_hbm,
              alsv, aldv, mhv, cntv, srcv, dstv, wslot, swv):
    c = lax.axis_index("c")
    s = lax.axis_index("s")
    base = s * TPB

    pltpu.sync_copy(als_hbm.at[c], alsv)
    pltpu.sync_copy(ald_hbm.at[c], aldv)
    pltpu.sync_copy(mh_hbm.at[c], mhv)
    pltpu.sync_copy(cnt_hbm.at[c, s], cntv)
    mh = mhv[0][0]

    def _sw(g, _):
        rbase = base + g * 16

        @pl.when(rbase < N)
        def _real():
            a16 = alsv[pl.ds(rbase, 16)]
            d16 = aldv[pl.ds(rbase, 16)]
            swv[pl.ds(g * 16, 16)] = jnp.exp(_leaky(a16 + d16) - mh)

        return 0
    lax.fori_loop(0, TPB // 16, _sw, 0)
    pltpu.sync_copy(swv, selfw_hbm.at[c, pl.ds(base, TPB)])

    def _slot(g, _):
        cnt = cntv[g][0]

        @pl.when(cnt <= TIER)
        def _ld_small():
            pltpu.sync_copy(lsrc_hbm.at[c, s, g, pl.ds(0, TIER)],
                            srcv.at[pl.ds(0, TIER)])
            pltpu.sync_copy(ldst_hbm.at[c, s, g, pl.ds(0, TIER)],
                            dstv.at[pl.ds(0, TIER)])

        @pl.when(cnt > TIER)
        def _ld_big():
            pltpu.sync_copy(lsrc_hbm.at[c, s, g], srcv)
            pltpu.sync_copy(ldst_hbm.at[c, s, g], dstv)

        def _g16(i, _):
            s16 = srcv[pl.ds(i * 16, 16)]
            d16 = dstv[pl.ds(i * 16, 16)]
            aw = plsc.load_gather(alsv, [s16])
            dw = plsc.load_gather(aldv, [d16])
            wslot[pl.ds(i * 16, 16)] = jnp.exp(_leaky(aw + dw) - mh)
            return 0
        lax.fori_loop(0, (cnt + 15) // 16, _g16, 0)

        @pl.when(cnt <= TIER)
        def _st_small():
            pltpu.sync_copy(wslot.at[pl.ds(0, TIER)],
                            wl_hbm.at[c, s, g, pl.ds(0, TIER)])

        @pl.when(cnt > TIER)
        def _st_big():
            pltpu.sync_copy(wslot, wl_hbm.at[c, s, g])
        return 0

    lax.fori_loop(0, NCHUNK, _slot, 0)


_wgt = functools.partial(
    pl.kernel,
    out_type=(
        jax.ShapeDtypeStruct((NB, NSC, NCHUNK, CP), jnp.float32),
        jax.ShapeDtypeStruct((NB, NP), jnp.float32),
    ),
    mesh=_MESH,
    scratch_types=[
        pltpu.VMEM((N,), jnp.float32),
        pltpu.VMEM((N,), jnp.float32),
        pltpu.VMEM((8, 16), jnp.float32),
        pltpu.VMEM((256, 16), jnp.int32),
        pltpu.VMEM((CP,), jnp.int32),
        pltpu.VMEM((CP,), jnp.int32),
        pltpu.VMEM((CP,), jnp.float32),
        pltpu.VMEM((TPB,), jnp.float32),
    ],
    compiler_params=pltpu.CompilerParams(needs_layout_passes=False),
)(_wgt_body)


def _agg_body(hf_hbm, selfw_hbm, lsrc_hbm, ldst_hbm, wl_hbm, cnt_hbm,
              num_hbm, den_hbm,
              cntv, srcv, dstv, wslot, adjv, rowsb, denstage, acc, dens,
              gsem0, gsem1):
    c = lax.axis_index("c")
    s = lax.axis_index("s")
    base = s * TPB
    cn = c * N
    lane = lax.iota(jnp.int32, 16)
    fz = jnp.zeros((16,), jnp.float32)

    pltpu.sync_copy(cnt_hbm.at[c, s], cntv)
    pltpu.sync_copy(selfw_hbm.at[c, pl.ds(base, TPB)], denstage)

    def _zd(i, _):
        dens[i] = 0.0
        return 0
    lax.fori_loop(0, TPB, _zd, 0)

    def _init(g, _):
        rbase = base + g * 16

        @pl.when(rbase < N)
        def _real():
            pltpu.sync_copy(hf_hbm.at[pl.ds(cn + rbase, 16)],
                            rowsb.at[0, pl.ds(0, 16)])

            def _row(r, _):
                row = g * 16 + r
                wvec = plsc.load_gather(denstage,
                                        [jnp.full((16,), row, jnp.int32)])
                for j in range(8):
                    sl = pl.ds(j * 16, 16)
                    acc[row, sl] = wvec * rowsb[0, r, sl]
                return 0
            lax.fori_loop(0, 16, _row, 0)

        return 0
    lax.fori_loop(0, TPB // 16, _init, 0)

    def _prep_adj(k, par):
        eb = k * SB
        for gg in range(SB // 16):
            s16 = srcv[pl.ds(eb + gg * 16, 16)]
            adjv[par, pl.ds(gg * 16, 16)] = s16 + cn

    def _fire_gather(par):
        gsem = gsem0 if par == 0 else gsem1
        pltpu.async_copy(hf_hbm.at[adjv.at[par]], rowsb.at[par], gsem)

    def _wait_gather(par):
        gsem = gsem0 if par == 0 else gsem1
        pltpu.make_async_copy(hf_hbm.at[adjv.at[par]], rowsb.at[par],
                              gsem).wait()

    def _proc_sub(k, par, cnt):
        eb = k * SB
        m = jnp.minimum(cnt - eb, SB)

        def _grp(g2, _):
            gb = g2 * 16
            d16 = dstv[pl.ds(eb + gb, 16)]
            w16 = wslot[pl.ds(eb + gb, 16)]
            w16 = jnp.where(gb + lane < m, w16, 0.0)
            dl16 = jnp.clip(d16 - base, 0, TPB - 1)
            for kk in range(16):
                d_loc = dl16[kk]
                w = w16[kk]
                dens[d_loc] = dens[d_loc] + w
                row_i = gb + kk
                for j in range(8):
                    sl = pl.ds(j * 16, 16)
                    acc[d_loc, sl] = acc[d_loc, sl] + w * rowsb[par, row_i, sl]
            return 0
        lax.fori_loop(0, (m + 15) // 16, _grp, 0)

    def _slot(g, _):
        cnt = cntv[g][0]

        @pl.when(cnt <= TIER)
        def _ld_small():
            pltpu.sync_copy(lsrc_hbm.at[c, s, g, pl.ds(0, TIER)],
                            srcv.at[pl.ds(0, TIER)])
            pltpu.sync_copy(ldst_hbm.at[c, s, g, pl.ds(0, TIER)],
                            dstv.at[pl.ds(0, TIER)])
            pltpu.sync_copy(wl_hbm.at[c, s, g, pl.ds(0, TIER)],
                            wslot.at[pl.ds(0, TIER)])

        @pl.when(cnt > TIER)
        def _ld_big():
            pltpu.sync_copy(lsrc_hbm.at[c, s, g], srcv)
            pltpu.sync_copy(ldst_hbm.at[c, s, g], dstv)
            pltpu.sync_copy(wl_hbm.at[c, s, g], wslot)

        nsub = (cnt + (SB - 1)) // SB

        @pl.when(nsub > 0)
        def _has():
            _prep_adj(0, 0)
            _fire_gather(0)

            def _pair(k2, _):
                for par in range(2):
                    cur = k2 * 2 + par

                    @pl.when(cur < nsub)
                    def _do(cur=cur, par=par):
                        @pl.when(cur + 1 < nsub)
                        def _pf():
                            _prep_adj(cur + 1, 1 - par)
                            _fire_gather(1 - par)

                        _wait_gather(par)
                        _proc_sub(cur, par, cnt)
                return 0
            lax.fori_loop(0, (nsub + 1) // 2, _pair, 0)
        return 0

    lax.fori_loop(0, NCHUNK, _slot, 0)

    def _wout(g, _):
        vec = fz
        for k in range(16):
            vec = jnp.where(lane == k, dens[g * 16 + k], vec)
        denstage[pl.ds(g * 16, 16)] = denstage[pl.ds(g * 16, 16)] + vec
        return 0
    lax.fori_loop(0, TPB // 16, _wout, 0)

    pltpu.sync_copy(acc, num_hbm.at[c, pl.ds(base, TPB)])
    pltpu.sync_copy(denstage, den_hbm.at[c, pl.ds(base, TPB)])


_agg = functools.partial(
    pl.kernel,
    out_type=(
        jax.ShapeDtypeStruct((NB, NP, F), jnp.float32),
        jax.ShapeDtypeStruct((NB, NP), jnp.float32),
    ),
    mesh=_MESH,
    scratch_types=[
        pltpu.VMEM((256, 16), jnp.int32),
        pltpu.VMEM((CP,), jnp.int32),
        pltpu.VMEM((CP,), jnp.int32),
        pltpu.VMEM((CP,), jnp.float32),
        pltpu.VMEM((2, SB), jnp.int32),
        pltpu.VMEM((2, SB, F), jnp.float32),
        pltpu.VMEM((TPB,), jnp.float32),
        pltpu.VMEM((TPB, F), jnp.float32),
        pltpu.SMEM((TPB,), jnp.float32),
        pltpu.SemaphoreType.DMA,
        pltpu.SemaphoreType.DMA,
    ],
    compiler_params=pltpu.CompilerParams(needs_layout_passes=False),
)(_agg_body)


BM = 1000
NBLK = N // BM


def _layer0_body(act_ref, w_ref, asrc_ref, adst_ref,
                 h_ref, als_ref, ald_ref, mh_ref, msc):
    i = pl.program_id(1)
    a = act_ref[0]
    h = lax.dot_general(a, w_ref[0], (((1,), (1,)), ((), ())),
                        preferred_element_type=jnp.float32)
    h_ref[0] = h
    als = h @ asrc_ref[0, 0]
    ald = h @ adst_ref[0, 0]
    als_ref[0, :, 0] = als
    ald_ref[0, :, 0] = ald
    ms = jnp.max(als)
    md = jnp.max(ald)

    @pl.when(i == 0)
    def _():
        msc[0] = ms
        msc[1] = md

    @pl.when(i > 0)
    def _():
        msc[0] = jnp.maximum(msc[0], ms)
        msc[1] = jnp.maximum(msc[1], md)

    @pl.when(i == NBLK - 1)
    def _():
        mh_ref[0] = jnp.full((8, 16), _leaky(msc[0] + msc[1]), jnp.float32)


def _layerN_body(num_ref, den_ref, bprev_ref, w_ref, asrc_ref, adst_ref,
                 act_ref, h_ref, als_ref, ald_ref, mh_ref, msc):
    i = pl.program_id(1)
    a = jax.nn.relu(num_ref[0] / (den_ref[0] + 1e-16) + bprev_ref[0])
    act_ref[0] = a
    h = lax.dot_general(a, w_ref[0], (((1,), (1,)), ((), ())),
                        preferred_element_type=jnp.float32)
    h_ref[0] = h
    als = h @ asrc_ref[0, 0]
    ald = h @ adst_ref[0, 0]
    als_ref[0, :, 0] = als
    ald_ref[0, :, 0] = ald
    ms = jnp.max(als)
    md = jnp.max(ald)

    @pl.when(i == 0)
    def _():
        msc[0] = ms
        msc[1] = md

    @pl.when(i > 0)
    def _():
        msc[0] = jnp.maximum(msc[0], ms)
        msc[1] = jnp.maximum(msc[1], md)

    @pl.when(i == NBLK - 1)
    def _():
        mh_ref[0] = jnp.full((8, 16), _leaky(msc[0] + msc[1]), jnp.float32)


_common_out = (
    jax.ShapeDtypeStruct((NB, N, F), jnp.float32),
    jax.ShapeDtypeStruct((NB, N, 1), jnp.float32),
    jax.ShapeDtypeStruct((NB, N, 1), jnp.float32),
    jax.ShapeDtypeStruct((NB, 8, 16), jnp.float32),
)
_common_out_specs = [
    pl.BlockSpec((1, BM, F), lambda b, i: (b, i, 0)),
    pl.BlockSpec((1, BM, 1), lambda b, i: (b, i, 0)),
    pl.BlockSpec((1, BM, 1), lambda b, i: (b, i, 0)),
    pl.BlockSpec((1, 8, 16), lambda b, i: (b, 0, 0)),
]
_w_specs = [
    pl.BlockSpec((1, F, F), lambda b, i: (b, 0, 0)),
    pl.BlockSpec((1, 1, F), lambda b, i: (b, 0, 0)),
    pl.BlockSpec((1, 1, F), lambda b, i: (b, 0, 0)),
]


def _layer0(act, W, asrc, adst):
    return pl.pallas_call(
        _layer0_body,
        grid=(NB, NBLK),
        in_specs=[pl.BlockSpec((1, BM, F), lambda b, i: (b, i, 0))] + _w_specs,
        out_specs=_common_out_specs,
        out_shape=_common_out,
        scratch_shapes=[pltpu.SMEM((2,), jnp.float32)],
    )(act, W, asrc, adst)


def _layerN(num, den, bprev, W, asrc, adst):
    return pl.pallas_call(
        _layerN_body,
        grid=(NB, NBLK),
        in_specs=[
            pl.BlockSpec((1, BM, F), lambda b, i: (b, i, 0)),
            pl.BlockSpec((1, BM, 1), lambda b, i: (b, i, 0)),
            pl.BlockSpec((1, 1, F), lambda b, i: (b, 0, 0)),
        ] + _w_specs,
        out_specs=[pl.BlockSpec((1, BM, F), lambda b, i: (b, i, 0))]
        + _common_out_specs,
        out_shape=(jax.ShapeDtypeStruct((NB, N, F), jnp.float32),)
        + _common_out,
        scratch_shapes=[pltpu.SMEM((2,), jnp.float32)],
    )(num, den, bprev, W, asrc, adst)


def _combine_body(a1_ref, a2_ref, num_ref, den_ref, b2_ref, cwt_ref, cb_ref,
                  z_ref):
    a3 = jax.nn.relu(num_ref[0] / (den_ref[0] + 1e-16) + b2_ref[0])
    z = (jnp.dot(a1_ref[0], cwt_ref[0, 0], preferred_element_type=jnp.float32)
         + jnp.dot(a2_ref[0], cwt_ref[0, 1], preferred_element_type=jnp.float32)
         + jnp.dot(a3, cwt_ref[0, 2], preferred_element_type=jnp.float32))
    z_ref[0] = z + cb_ref[0]


def _combine(a1, a2, num, den, b2, cwt, cb):
    return pl.pallas_call(
        _combine_body,
        grid=(NB, NBLK),
        in_specs=[
            pl.BlockSpec((1, BM, F), lambda b, i: (b, i, 0)),
            pl.BlockSpec((1, BM, F), lambda b, i: (b, i, 0)),
            pl.BlockSpec((1, BM, F), lambda b, i: (b, i, 0)),
            pl.BlockSpec((1, BM, 1), lambda b, i: (b, i, 0)),
            pl.BlockSpec((1, 1, F), lambda b, i: (b, 0, 0)),
            pl.BlockSpec((1, L, F, F), lambda b, i: (b, 0, 0, 0)),
            pl.BlockSpec((1, 1, F), lambda b, i: (b, 0, 0)),
        ],
        out_specs=pl.BlockSpec((1, BM, F), lambda b, i: (b, i, 0)),
        out_shape=jax.ShapeDtypeStruct((NB, N, F), jnp.float32),
    )(a1, a2, num, den, b2, cwt, cb)


def _final_body(x_ref, y_ref, o_ref):
    o_ref[...] = lax.dot_general(
        x_ref[...], y_ref[...], (((1,), (1,)), ((), ())),
        preferred_element_type=jnp.float32)


def _final(x, y, bm=400):
    return pl.pallas_call(
        _final_body,
        grid=(N // bm,),
        in_specs=[
            pl.BlockSpec((bm, F), lambda i: (i, 0)),
            pl.BlockSpec((N, F), lambda i: (0, 0)),
        ],
        out_specs=pl.BlockSpec((bm, N), lambda i: (i, 0)),
        out_shape=jax.ShapeDtypeStruct((N, N), jnp.float32),
    )(x, y)


def kernel(x_m, x_d, gx_W, gx_as, gx_ad, gx_b, gy_W, gy_as, gy_ad, gy_b,
           cnn_x_w, cnn_x_b, cnn_y_w, cnn_y_b, mm_f_edges, dd_f_edges):
    act0 = jnp.stack([x_m, x_d])
    W = jnp.stack([gx_W, gy_W])
    asrc = jnp.stack([gx_as, gy_as])[:, :, None, :]
    adst = jnp.stack([gx_ad, gy_ad])[:, :, None, :]
    bias = jnp.stack([gx_b, gy_b])[:, :, None, :]
    src = jnp.stack([mm_f_edges[0], dd_f_edges[0]])
    dst = jnp.stack([mm_f_edges[1], dd_f_edges[1]])
    cwt = jnp.stack([
        jnp.transpose(cnn_x_w[..., 0], (1, 2, 0)),
        jnp.transpose(cnn_y_w[..., 0], (1, 2, 0)),
    ])
    cb = jnp.stack([cnn_x_b, cnn_y_b])[:, None, :]

    lsrc, ldst, cnt = _bucket(src, dst)

    acts = []
    num = den = None
    for l in range(L):
        if l == 0:
            h, als, ald, mh = _layer0(act0, W[:, 0], asrc[:, 0], adst[:, 0])
        else:
            act, h, als, ald, mh = _layerN(
                num, den[..., None], bias[:, l - 1], W[:, l],
                asrc[:, l], adst[:, l])
            acts.append(act)
        hf = jnp.reshape(h, (NB * N, F))
        wl, selfw = _wgt(jnp.reshape(als, (NB, N)), jnp.reshape(ald, (NB, N)),
                         mh, lsrc, ldst, cnt)
        num, den = _agg(hf, selfw, lsrc, ldst, wl, cnt)
        num = num[:, :N]
        den = den[:, :N]

    z = _combine(acts[0], acts[1], num, den[..., None],
                 bias[:, L - 1], cwt, cb)
    return _final(z[0], z[1])
